# Initial kernel scaffold; baseline (speedup 1.0000x reference)
import jax
import jax.numpy as jnp
from jax.experimental import pallas as pl


def kernel(x, embedding, positional_encoding):
    raise NotImplementedError("write your pallas kernel here")


# speedup vs baseline: 2.1023x; 2.1023x over previous
import functools

import jax
import jax.numpy as jnp
from jax import lax
from jax.experimental import pallas as pl
from jax.experimental.pallas import tpu as pltpu
from jax.experimental.pallas import tpu_sc as plsc

LANES = 16
NUM_CORES = 2
NUM_SUBCORES = 16
NW = NUM_CORES * NUM_SUBCORES
R = 128


def _encode(xf, pos, embedding, total, S, D):
    n_chunks_w = (total // NW) // R
    b_per_w = total // NW

    mesh = plsc.VectorSubcoreMesh(core_axis_name="c", subcore_axis_name="s")

    @functools.partial(
        pl.kernel,
        out_type=jax.ShapeDtypeStruct((total, D), jnp.float32),
        mesh=mesh,
        scratch_types=[
            pltpu.VMEM((n_chunks_w, R), jnp.int32),
            pltpu.VMEM((2 * S, D), jnp.float32),
            pltpu.VMEM((R, D), jnp.float32),
            pltpu.SemaphoreType.DMA,
        ],
    )
    def enc(x_hbm, pos_hbm, emb_hbm, out_hbm, idx_v, pos2_v, rows_v, sem):
        wid = lax.axis_index("s") * NUM_CORES + lax.axis_index("c")
        pltpu.sync_copy(x_hbm.at[pl.ds(wid * n_chunks_w, n_chunks_w)], idx_v)
        pltpu.sync_copy(pos_hbm, pos2_v.at[pl.ds(0, S)])
        pltpu.sync_copy(pos_hbm, pos2_v.at[pl.ds(S, S)])

        def chunk(c, carry):
            pltpu.async_copy(emb_hbm.at[idx_v.at[c]], rows_v, sem).wait()
            p = lax.rem(c * R, S)

            def addrow(r, carry2):
                for j in range(D // LANES):
                    sl = pl.ds(j * LANES, LANES)
                    rows_v[r, sl] = rows_v[r, sl] + pos2_v[p + r, sl]
                return carry2

            lax.fori_loop(0, R, addrow, 0, unroll=2)
            pltpu.sync_copy(
                rows_v, out_hbm.at[pl.ds(wid * b_per_w + c * R, R)]
            )
            return carry

        lax.fori_loop(0, n_chunks_w, chunk, 0)

    return enc(xf, pos, embedding)


def kernel(x, embedding, positional_encoding):
    B, S = x.shape
    V, D = embedding.shape
    total = B * S
    xf = x.reshape(total // R, R).astype(jnp.int32)
    pos = positional_encoding[:S]
    out = _encode(xf, pos, embedding, total, S, D)
    return out.reshape(B, S, D)


# speedup vs baseline: 2.8770x; 1.3685x over previous
import functools

import jax
import jax.numpy as jnp
from jax import lax
from jax.experimental import pallas as pl
from jax.experimental.pallas import tpu as pltpu
from jax.experimental.pallas import tpu_sc as plsc

LANES = 16
NUM_CORES = 2
NUM_SUBCORES = 16
NW = NUM_CORES * NUM_SUBCORES
R = 80
NBUF = 4


def _encode(xf, pos, embedding, total, S, D):
    b_per_w = total // NW
    n_chunks = b_per_w // R
    n_groups = n_chunks // NBUF
    PB = S + R - (S % R if S % R else R)
    nvr = D // LANES

    mesh = plsc.VectorSubcoreMesh(core_axis_name="c", subcore_axis_name="s")

    @functools.partial(
        pl.kernel,
        out_type=jax.ShapeDtypeStruct((total, D), jnp.float32),
        mesh=mesh,
        scratch_types=[
            pltpu.VMEM((n_chunks, R), jnp.int32),
            pltpu.VMEM((PB, D), jnp.float32),
            pltpu.VMEM((NBUF, R, D), jnp.float32),
        ]
        + [pltpu.SemaphoreType.DMA] * (2 * NBUF),
    )
    def enc(x_hbm, pos_hbm, emb_hbm, out_hbm, idx_v, pos2_v, rows_v, *sems):
        gsem = sems[:NBUF]
        ssem = sems[NBUF:]
        wid = lax.axis_index("s") * NUM_CORES + lax.axis_index("c")
        row0 = wid * b_per_w

        pltpu.sync_copy(x_hbm.at[pl.ds(wid * n_chunks, n_chunks)], idx_v)
        pltpu.sync_copy(pos_hbm, pos2_v.at[pl.ds(0, S)])
        pltpu.sync_copy(pos_hbm.at[pl.ds(0, PB - S)], pos2_v.at[pl.ds(S, PB - S)])

        for k in range(2):
            pltpu.async_copy(emb_hbm.at[idx_v.at[k]], rows_v.at[k], gsem[k])

        def group(g, carry):
            for u in range(NBUF):
                c = g * NBUF + u
                rb = rows_v.at[u]
                pltpu.make_async_copy(
                    emb_hbm.at[idx_v.at[c]], rb, gsem[u]
                ).wait()

                p = lax.rem(c * R, S)

                def addrow(r, carry2, rb=rb, p=p):
                    for j in range(nvr):
                        sl = pl.ds(j * LANES, LANES)
                        rb[r, sl] = rb[r, sl] + pos2_v[p + r, sl]
                    return carry2

                lax.fori_loop(0, R, addrow, 0, unroll=2)

                b2 = (u + 2) % NBUF
                ob2 = out_hbm.at[pl.ds(row0 + (c - 2) * R, R)]

                @pl.when(c >= 2)
                def _(b2=b2, ob2=ob2):
                    pltpu.make_async_copy(rows_v.at[b2], ob2, ssem[b2]).wait()

                pltpu.async_copy(
                    rb, out_hbm.at[pl.ds(row0 + c * R, R)], ssem[u]
                )

                @pl.when(c + 2 < n_chunks)
                def _(c=c, b2=b2):
                    pltpu.async_copy(
                        emb_hbm.at[idx_v.at[c + 2]], rows_v.at[b2], gsem[b2]
                    )

            return carry

        lax.fori_loop(0, n_groups, group, 0)

        for c in (n_chunks - 2, n_chunks - 1):
            b = c % NBUF
            pltpu.make_async_copy(
                rows_v.at[b], out_hbm.at[pl.ds(row0 + c * R, R)], ssem[b]
            ).wait()

    return enc(xf, pos, embedding)


def kernel(x, embedding, positional_encoding):
    B, S = x.shape
    V, D = embedding.shape
    total = B * S
    xf = x.reshape(total // R, R).astype(jnp.int32)
    pos = positional_encoding[:S]
    out = _encode(xf, pos, embedding, total, S, D)
    return out.reshape(B, S, D)


# speedup vs baseline: 8.9580x; 3.1137x over previous
import functools

import jax
import jax.numpy as jnp
from jax import lax
from jax.experimental import pallas as pl
from jax.experimental.pallas import tpu as pltpu
from jax.experimental.pallas import tpu_sc as plsc

LANES = 16
NUM_CORES = 2
NUM_SUBCORES = 16
NW = NUM_CORES * NUM_SUBCORES
R = 80
NBUF = 5
GA = NBUF - 2


def _encode(xf, pos, embedding, total, S, D):
    b_per_w = total // NW
    n_chunks = b_per_w // R
    n_groups = n_chunks // NBUF
    PB = S + R - (S % R if S % R else R)
    nvr = D // LANES

    mesh = plsc.VectorSubcoreMesh(core_axis_name="c", subcore_axis_name="s")

    @functools.partial(
        pl.kernel,
        out_type=jax.ShapeDtypeStruct((total, D), jnp.float32),
        mesh=mesh,
        scratch_types=[
            pltpu.VMEM((n_chunks, R), jnp.int32),
            pltpu.VMEM((PB, D), jnp.float32),
            pltpu.VMEM((NBUF, R, D), jnp.float32),
        ]
        + [pltpu.SemaphoreType.DMA] * (2 * NBUF),
    )
    def enc(x_hbm, pos_hbm, emb_hbm, out_hbm, idx_v, pos2_v, rows_v, *sems):
        gsem = sems[:NBUF]
        ssem = sems[NBUF:]
        wid = lax.axis_index("s") * NUM_CORES + lax.axis_index("c")
        row0 = wid * b_per_w

        pltpu.sync_copy(x_hbm.at[pl.ds(wid * n_chunks, n_chunks)], idx_v)
        pltpu.sync_copy(pos_hbm, pos2_v.at[pl.ds(0, S)])
        pltpu.sync_copy(pos_hbm.at[pl.ds(0, PB - S)], pos2_v.at[pl.ds(S, PB - S)])

        for k in range(GA):
            pltpu.async_copy(emb_hbm.at[idx_v.at[k]], rows_v.at[k], gsem[k])

        def group(g, carry):
            for u in range(NBUF):
                c = g * NBUF + u
                rb = rows_v.at[u]
                pltpu.make_async_copy(
                    emb_hbm.at[idx_v.at[c]], rb, gsem[u]
                ).wait()

                p = lax.rem(c * R, S)

                def addrow(r, carry2, rb=rb, p=p):
                    pvals = [
                        pos2_v[p + r, pl.ds(j * LANES, LANES)]
                        for j in range(nvr)
                    ]
                    for j in range(nvr):
                        plsc.addupdate(
                            rb.at[r, pl.ds(j * LANES, LANES)], pvals[j]
                        )
                    return carry2

                lax.fori_loop(0, R, addrow, 0, unroll=2)

                b2 = (u + GA) % NBUF
                ob2 = out_hbm.at[pl.ds(row0 + (c - 2) * R, R)]

                @pl.when(c >= 2)
                def _(b2=b2, ob2=ob2):
                    pltpu.make_async_copy(rows_v.at[b2], ob2, ssem[b2]).wait()

                pltpu.async_copy(
                    rb, out_hbm.at[pl.ds(row0 + c * R, R)], ssem[u]
                )

                @pl.when(c + GA < n_chunks)
                def _(c=c, b2=b2):
                    pltpu.async_copy(
                        emb_hbm.at[idx_v.at[c + GA]], rows_v.at[b2], gsem[b2]
                    )

            return carry

        lax.fori_loop(0, n_groups, group, 0)

        for c in (n_chunks - 2, n_chunks - 1):
            b = c % NBUF
            pltpu.make_async_copy(
                rows_v.at[b], out_hbm.at[pl.ds(row0 + c * R, R)], ssem[b]
            ).wait()

    return enc(xf, pos, embedding)


def kernel(x, embedding, positional_encoding):
    B, S = x.shape
    V, D = embedding.shape
    total = B * S
    xf = x.reshape(total // R, R).astype(jnp.int32)
    pos = positional_encoding[:S]
    out = _encode(xf, pos, embedding, total, S, D)
    return out.reshape(B, S, D)
